# baseline (speedup 1.0000x reference)
import functools

import jax
import jax.numpy as jnp
from jax import lax
from jax.experimental import pallas as pl
from jax.experimental.pallas import tpu as pltpu
from jax.experimental.pallas import tpu_sc as plsc

B = 16384
HIDDEN = 128
HALF = 64
W2V = 10
NPOS = 32
VOCAB = 100000
GR = VOCAB * W2V // 16
NC = 2
NS = 16
NW = NC * NS
CHUNK = B // NW
HCHUNK = CHUNK // 2
L = 16


def _fused_rows(pv_ref, ov_ref, t0, ptab_ref, raw_ref, w_ref, b_ref,
                out_ref):
    wvals = [[w_ref[k, L * j:L * (j + 1)] for j in range(HALF // L)]
             for k in range(W2V)]
    bvals = [b_ref[L * j:L * (j + 1)] for j in range(HALF // L)]
    iota = jnp.arange(L, dtype=jnp.int32)
    zero = jnp.zeros((L,), dtype=jnp.int32)

    @plsc.parallel_loop(0, HCHUNK, 1, unroll=2)
    def body(t):
        idx_t = jnp.full((L,), t, dtype=jnp.int32)
        p_t = plsc.load_gather(pv_ref, [idx_t + t0])
        for j in range(HALF // L):
            out_ref[t, L * j:L * (j + 1)] = plsc.load_gather(
                ptab_ref, [p_t, iota + L * j])
        fl = plsc.load_gather(ov_ref, [idx_t + t0]) + (2 * L) * t
        accs = list(bvals)
        for k in range(W2V):
            rk = plsc.load_gather(raw_ref, [zero, fl + k])
            accs = [a + rk * wvals[k][j] for j, a in enumerate(accs)]
        for j in range(HALF // L):
            out_ref[t, HALF + L * j:HALF + L * (j + 1)] = accs[j]


@functools.partial(
    pl.kernel,
    out_type=(
        jax.ShapeDtypeStruct((B, HIDDEN), jnp.float32),
        jax.ShapeDtypeStruct((B, HIDDEN), jnp.float32),
    ),
    mesh=plsc.VectorSubcoreMesh(core_axis_name="c", subcore_axis_name="s",
                                num_cores=NC, num_subcores=NS),
    compiler_params=pltpu.CompilerParams(use_tc_tiling_on_sc=False,
                                         needs_layout_passes=False),
    scratch_types=[
        pltpu.VMEM((CHUNK,), jnp.int32),
        pltpu.VMEM((CHUNK,), jnp.int32),
        pltpu.VMEM((CHUNK,), jnp.int32),
        pltpu.VMEM((2 * CHUNK,), jnp.int32),
        pltpu.VMEM((CHUNK,), jnp.int32),
        pltpu.VMEM((NPOS, HALF), jnp.float32),
        pltpu.VMEM((NPOS, HALF), jnp.float32),
        pltpu.VMEM((2 * HCHUNK, L), jnp.float32),
        pltpu.VMEM((2 * HCHUNK, L), jnp.float32),
        pltpu.VMEM((2 * HCHUNK, L), jnp.float32),
        pltpu.VMEM((2 * HCHUNK, L), jnp.float32),
        pltpu.VMEM((HCHUNK, HIDDEN), jnp.float32),
        pltpu.VMEM((W2V, HALF), jnp.float32),
        pltpu.VMEM((W2V, HALF), jnp.float32),
        pltpu.VMEM((HALF,), jnp.float32),
        pltpu.VMEM((HALF,), jnp.float32),
        pltpu.SemaphoreType.DMA,
        pltpu.SemaphoreType.DMA,
        pltpu.SemaphoreType.DMA,
        pltpu.SemaphoreType.DMA,
        pltpu.SemaphoreType.DMA,
    ],
)
def _mixed_context_sc(x_hbm, pos_table_hbm, idx2ctx_hbm, pos_c_hbm,
                      pos_h_hbm, w2v_c_hbm, w2v_h_hbm, c_w_hbm, c_b_hbm,
                      h_w_hbm, h_b_hbm, out_c_hbm, out_h_hbm,
                      xv, pv, cv, gidx, ov, ptabc, ptabh,
                      rawc0, rawh0, rawc1, rawh1, outb,
                      wcv, whv, bcv, bhv, s0, s1, s2, s3, s4):
    wid = lax.axis_index("s") * NC + lax.axis_index("c")
    base = wid * CHUNK
    iota = jnp.arange(L, dtype=jnp.int32)

    pltpu.sync_copy(x_hbm.at[pl.ds(base, CHUNK)], xv)
    hp = pltpu.async_copy(pos_table_hbm.at[xv], pv, s0)
    hc = pltpu.async_copy(idx2ctx_hbm.at[xv], cv, s1)

    pltpu.sync_copy(pos_c_hbm, ptabc)
    pltpu.sync_copy(pos_h_hbm, ptabh)
    pltpu.sync_copy(c_w_hbm, wcv)
    pltpu.sync_copy(h_w_hbm, whv)
    pltpu.sync_copy(c_b_hbm, bcv)
    pltpu.sync_copy(h_b_hbm, bhv)

    hc.wait()
    for i in range(CHUNK // L):
        b10 = cv[pl.ds(i * L, L)] * W2V
        g0 = lax.shift_right_logical(b10, 4)
        g1 = jnp.minimum(g0 + 1, GR - 1)
        plsc.store_scatter(gidx, [iota * 2 + 2 * L * i], g0)
        plsc.store_scatter(gidx, [iota * 2 + 2 * L * i + 1], g1)
        ov[pl.ds(i * L, L)] = jnp.bitwise_and(b10, 15)
    hc0 = pltpu.async_copy(w2v_c_hbm.at[gidx.at[pl.ds(0, CHUNK)]], rawc0, s1)
    hh0 = pltpu.async_copy(w2v_h_hbm.at[gidx.at[pl.ds(0, CHUNK)]], rawh0, s2)
    hc1 = pltpu.async_copy(w2v_c_hbm.at[gidx.at[pl.ds(CHUNK, CHUNK)]],
                           rawc1, s3)
    hh1 = pltpu.async_copy(w2v_h_hbm.at[gidx.at[pl.ds(CHUNK, CHUNK)]],
                           rawh1, s4)
    hp.wait()

    hc0.wait()
    _fused_rows(pv, ov, 0, ptabc, rawc0, wcv, bcv, outb)
    pltpu.sync_copy(outb, out_c_hbm.at[pl.ds(base, HCHUNK)])
    hh0.wait()
    _fused_rows(pv, ov, 0, ptabh, rawh0, whv, bhv, outb)
    pltpu.sync_copy(outb, out_h_hbm.at[pl.ds(base, HCHUNK)])

    hc1.wait()
    _fused_rows(pv, ov, HCHUNK, ptabc, rawc1, wcv, bcv, outb)
    pltpu.sync_copy(outb, out_c_hbm.at[pl.ds(base + HCHUNK, HCHUNK)])
    hh1.wait()
    _fused_rows(pv, ov, HCHUNK, ptabh, rawh1, whv, bhv, outb)
    pltpu.sync_copy(outb, out_h_hbm.at[pl.ds(base + HCHUNK, HCHUNK)])


def kernel(x, pos_table, idx2context, pos_c_emb, pos_h_emb, w2v_c_emb,
           w2v_h_emb, c_lin_w, c_lin_b, h_lin_w, h_lin_b):
    w2v_c = w2v_c_emb.reshape(GR, L)
    w2v_h = w2v_h_emb.reshape(GR, L)
    out_c, out_h = _mixed_context_sc(
        x, pos_table, idx2context, pos_c_emb, pos_h_emb,
        w2v_c, w2v_h, c_lin_w, c_lin_b, h_lin_w, h_lin_b)
    return (out_c.reshape(1, B, HIDDEN), out_h.reshape(1, B, HIDDEN))


# speedup vs baseline: 1.7503x; 1.7503x over previous
import functools

import jax
import jax.numpy as jnp
from jax import lax
from jax.experimental import pallas as pl
from jax.experimental.pallas import tpu as pltpu
from jax.experimental.pallas import tpu_sc as plsc

B = 16384
HIDDEN = 128
HALF = 64
W2V = 10
W2VPAD = 32
NPOS = 32
NC = 2
NS = 16
NW = NC * NS
CHUNK = B // NW
HCHUNK = CHUNK // 2
L = 16


def _fused_rows(pv_ref, t0, ptab_ref, raw_ref, col0, w_ref, b_ref, out_ref):
    wvals = [[w_ref[k, L * j:L * (j + 1)] for j in range(HALF // L)]
             for k in range(W2V)]
    bvals = [b_ref[L * j:L * (j + 1)] for j in range(HALF // L)]
    iota = jnp.arange(L, dtype=jnp.int32)

    @plsc.parallel_loop(0, HCHUNK, 1, unroll=2)
    def body(t):
        idx_t = jnp.full((L,), t, dtype=jnp.int32)
        p_t = plsc.load_gather(pv_ref, [idx_t + t0])
        for j in range(HALF // L):
            out_ref[t, L * j:L * (j + 1)] = plsc.load_gather(
                ptab_ref, [p_t, iota + L * j])
        accs = list(bvals)
        for k in range(W2V):
            idx_k = jnp.full((L,), col0 + k, dtype=jnp.int32)
            rk = plsc.load_gather(raw_ref, [idx_t + t0, idx_k])
            accs = [a + rk * wvals[k][j] for j, a in enumerate(accs)]
        for j in range(HALF // L):
            out_ref[t, HALF + L * j:HALF + L * (j + 1)] = accs[j]


@functools.partial(
    pl.kernel,
    out_type=(
        jax.ShapeDtypeStruct((B, HIDDEN), jnp.float32),
        jax.ShapeDtypeStruct((B, HIDDEN), jnp.float32),
    ),
    mesh=plsc.VectorSubcoreMesh(core_axis_name="c", subcore_axis_name="s",
                                num_cores=NC, num_subcores=NS),
    compiler_params=pltpu.CompilerParams(use_tc_tiling_on_sc=False,
                                         needs_layout_passes=False),
    scratch_types=[
        pltpu.VMEM((CHUNK,), jnp.int32),
        pltpu.VMEM((CHUNK,), jnp.int32),
        pltpu.VMEM((CHUNK,), jnp.int32),
        pltpu.VMEM((NPOS, HALF), jnp.float32),
        pltpu.VMEM((NPOS, HALF), jnp.float32),
        pltpu.VMEM((CHUNK, W2VPAD), jnp.float32),
        pltpu.VMEM((HCHUNK, HIDDEN), jnp.float32),
        pltpu.VMEM((HCHUNK, HIDDEN), jnp.float32),
        pltpu.VMEM((W2V, HALF), jnp.float32),
        pltpu.VMEM((W2V, HALF), jnp.float32),
        pltpu.VMEM((HALF,), jnp.float32),
        pltpu.VMEM((HALF,), jnp.float32),
        pltpu.SemaphoreType.DMA,
        pltpu.SemaphoreType.DMA,
        pltpu.SemaphoreType.DMA,
        pltpu.SemaphoreType.DMA,
    ],
)
def _mixed_context_sc(x_hbm, pos_table_hbm, idx2ctx_hbm, pos_c_hbm,
                      pos_h_hbm, w2v_hbm, c_w_hbm, c_b_hbm,
                      h_w_hbm, h_b_hbm, out_c_hbm, out_h_hbm,
                      xv, pv, cv, ptabc, ptabh, rawv, outb0, outb1,
                      wcv, whv, bcv, bhv, s0, s1, s2, s3):
    wid = lax.axis_index("s") * NC + lax.axis_index("c")
    base = wid * CHUNK

    pltpu.sync_copy(x_hbm.at[pl.ds(base, CHUNK)], xv)
    hp = pltpu.async_copy(pos_table_hbm.at[xv], pv, s0)
    hc = pltpu.async_copy(idx2ctx_hbm.at[xv], cv, s1)

    pltpu.sync_copy(pos_c_hbm, ptabc)
    pltpu.sync_copy(pos_h_hbm, ptabh)
    pltpu.sync_copy(c_w_hbm, wcv)
    pltpu.sync_copy(h_w_hbm, whv)
    pltpu.sync_copy(c_b_hbm, bcv)
    pltpu.sync_copy(h_b_hbm, bhv)

    hc.wait()
    hr0 = pltpu.async_copy(
        w2v_hbm.at[cv.at[pl.ds(0, HCHUNK)]], rawv.at[pl.ds(0, HCHUNK)], s1)
    hr1 = pltpu.async_copy(
        w2v_hbm.at[cv.at[pl.ds(HCHUNK, HCHUNK)]],
        rawv.at[pl.ds(HCHUNK, HCHUNK)], s2)
    hp.wait()

    hr0.wait()
    _fused_rows(pv, 0, ptabc, rawv, 0, wcv, bcv, outb0)
    oc0 = pltpu.async_copy(outb0, out_c_hbm.at[pl.ds(base, HCHUNK)], s0)
    _fused_rows(pv, 0, ptabh, rawv, W2V, whv, bhv, outb1)
    oh0 = pltpu.async_copy(outb1, out_h_hbm.at[pl.ds(base, HCHUNK)], s3)

    hr1.wait()
    oc0.wait()
    _fused_rows(pv, HCHUNK, ptabc, rawv, 0, wcv, bcv, outb0)
    oc1 = pltpu.async_copy(
        outb0, out_c_hbm.at[pl.ds(base + HCHUNK, HCHUNK)], s0)
    oh0.wait()
    _fused_rows(pv, HCHUNK, ptabh, rawv, W2V, whv, bhv, outb1)
    oh1 = pltpu.async_copy(
        outb1, out_h_hbm.at[pl.ds(base + HCHUNK, HCHUNK)], s3)

    oc1.wait()
    oh1.wait()


def kernel(x, pos_table, idx2context, pos_c_emb, pos_h_emb, w2v_c_emb,
           w2v_h_emb, c_lin_w, c_lin_b, h_lin_w, h_lin_b):
    w2v = jnp.concatenate(
        [w2v_c_emb, w2v_h_emb,
         jnp.zeros((w2v_c_emb.shape[0], W2VPAD - 2 * W2V), jnp.float32)],
        axis=1)
    out_c, out_h = _mixed_context_sc(
        x, pos_table, idx2context, pos_c_emb, pos_h_emb,
        w2v, c_lin_w, c_lin_b, h_lin_w, h_lin_b)
    return (out_c.reshape(1, B, HIDDEN), out_h.reshape(1, B, HIDDEN))


# speedup vs baseline: 1.8732x; 1.0702x over previous
import functools

import jax
import jax.numpy as jnp
from jax import lax
from jax.experimental import pallas as pl
from jax.experimental.pallas import tpu as pltpu
from jax.experimental.pallas import tpu_sc as plsc

B = 16384
HIDDEN = 128
HALF = 64
W2V = 10
W2VPAD = 32
NPOS = 32
NC = 2
NS = 16
NW = NC * NS
CHUNK = B // NW
L = 16


def _fused_rows(pv_ref, ptab_ref, raw_ref, col0, w_ref, b_ref, out_ref):
    wvals = [[w_ref[k, L * j:L * (j + 1)] for j in range(HALF // L)]
             for k in range(W2V)]
    bvals = [b_ref[L * j:L * (j + 1)] for j in range(HALF // L)]
    iota = jnp.arange(L, dtype=jnp.int32)

    @plsc.parallel_loop(0, CHUNK, 1, unroll=2)
    def body(t):
        idx_t = jnp.full((L,), t, dtype=jnp.int32)
        p_t = plsc.load_gather(pv_ref, [idx_t])
        for j in range(HALF // L):
            out_ref[t, L * j:L * (j + 1)] = plsc.load_gather(
                ptab_ref, [p_t, iota + L * j])
        accs = list(bvals)
        for k in range(W2V):
            idx_k = jnp.full((L,), col0 + k, dtype=jnp.int32)
            rk = plsc.load_gather(raw_ref, [idx_t, idx_k])
            accs = [a + rk * wvals[k][j] for j, a in enumerate(accs)]
        for j in range(HALF // L):
            out_ref[t, HALF + L * j:HALF + L * (j + 1)] = accs[j]


@functools.partial(
    pl.kernel,
    out_type=(
        jax.ShapeDtypeStruct((B, HIDDEN), jnp.float32),
        jax.ShapeDtypeStruct((B, HIDDEN), jnp.float32),
    ),
    mesh=plsc.VectorSubcoreMesh(core_axis_name="c", subcore_axis_name="s",
                                num_cores=NC, num_subcores=NS),
    compiler_params=pltpu.CompilerParams(use_tc_tiling_on_sc=False,
                                         needs_layout_passes=False),
    scratch_types=[
        pltpu.VMEM((CHUNK,), jnp.int32),
        pltpu.VMEM((CHUNK,), jnp.int32),
        pltpu.VMEM((CHUNK,), jnp.int32),
        pltpu.VMEM((NPOS, HALF), jnp.float32),
        pltpu.VMEM((NPOS, HALF), jnp.float32),
        pltpu.VMEM((CHUNK, W2VPAD), jnp.float32),
        pltpu.VMEM((CHUNK, HIDDEN), jnp.float32),
        pltpu.VMEM((W2V, HALF), jnp.float32),
        pltpu.VMEM((W2V, HALF), jnp.float32),
        pltpu.VMEM((HALF,), jnp.float32),
        pltpu.VMEM((HALF,), jnp.float32),
        pltpu.SemaphoreType.DMA,
        pltpu.SemaphoreType.DMA,
        pltpu.SemaphoreType.DMA,
    ],
)
def _mixed_context_sc(x_hbm, pos_table_hbm, idx2ctx_hbm, pos_c_hbm,
                      pos_h_hbm, w2v_hbm, c_w_hbm, c_b_hbm,
                      h_w_hbm, h_b_hbm, out_c_hbm, out_h_hbm,
                      xv, pv, cv, ptabc, ptabh, rawv, outb,
                      wcv, whv, bcv, bhv, s0, s1, s2):
    wid = lax.axis_index("s") * NC + lax.axis_index("c")
    base = wid * CHUNK

    pltpu.sync_copy(x_hbm.at[pl.ds(base, CHUNK)], xv)
    hp = pltpu.async_copy(pos_table_hbm.at[xv], pv, s0)
    hc = pltpu.async_copy(idx2ctx_hbm.at[xv], cv, s1)

    h1 = pltpu.async_copy(pos_c_hbm, ptabc, s2)
    h2 = pltpu.async_copy(pos_h_hbm, ptabh, s2)
    h3 = pltpu.async_copy(c_w_hbm, wcv, s2)
    h4 = pltpu.async_copy(h_w_hbm, whv, s2)
    h5 = pltpu.async_copy(c_b_hbm, bcv, s2)
    h6 = pltpu.async_copy(h_b_hbm, bhv, s2)

    hc.wait()
    hr = pltpu.async_copy(w2v_hbm.at[cv], rawv, s1)
    hp.wait()
    h6.wait()
    h5.wait()
    h4.wait()
    h3.wait()
    h2.wait()
    h1.wait()
    hr.wait()

    _fused_rows(pv, ptabc, rawv, 0, wcv, bcv, outb)
    pltpu.sync_copy(outb, out_c_hbm.at[pl.ds(base, CHUNK)])
    _fused_rows(pv, ptabh, rawv, W2V, whv, bhv, outb)
    pltpu.sync_copy(outb, out_h_hbm.at[pl.ds(base, CHUNK)])


def kernel(x, pos_table, idx2context, pos_c_emb, pos_h_emb, w2v_c_emb,
           w2v_h_emb, c_lin_w, c_lin_b, h_lin_w, h_lin_b):
    w2v = jnp.concatenate(
        [w2v_c_emb, w2v_h_emb,
         jnp.zeros((w2v_c_emb.shape[0], W2VPAD - 2 * W2V), jnp.float32)],
        axis=1)
    out_c, out_h = _mixed_context_sc(
        x, pos_table, idx2context, pos_c_emb, pos_h_emb,
        w2v, c_lin_w, c_lin_b, h_lin_w, h_lin_b)
    return (out_c.reshape(1, B, HIDDEN), out_h.reshape(1, B, HIDDEN))
